# Initial kernel scaffold; baseline (speedup 1.0000x reference)
import functools
import math

import jax
import jax.numpy as jnp
from jax import lax
from jax.experimental import pallas as pl
from jax.experimental.pallas import tpu as pltpu
from jax.experimental.pallas import tpu_sc as plsc

_NUM_NETS = 50000
_PPN = 4
_NUM_PINS = _NUM_NETS * _PPN
_NBX = 256
_NBY = 256
_XL, _YL, _XH, _YH = 0.0, 0.0, 1.0, 1.0
_ROUTING_H = 100.0
_ROUTING_V = 100.0
_MACRO_UTIL_H = 10.0
_MACRO_UTIL_V = 10.0
_NUM_MACROS = 200

_NW = 32
_CHUNK = 1568
_NETS_PAD = _NW * _CHUNK
_MACRO_PAD = 512
_KTOT = _NETS_PAD + _MACRO_PAD
_KT = 512
_G = _KTOT // _KT

_BSX = (_XH - _XL) / _NBX
_BSY = (_YH - _YL) / _NBY
_INV_CAPA_H = float(_NBX * _NBY) / _ROUTING_H
_INV_CAPA_V = float(_NBX * _NBY) / _ROUTING_V

_SIGMA = (1.0 / 16.0) * min((_XH - _XL) / _BSX, (_YH - _YL) / _BSY)
_pdf = [math.exp(-0.5 * (t / _SIGMA) ** 2) for t in (-1.0, 0.0, 1.0)]
_gs = sum(_pdf)
_G0, _G1, _G2 = (_pdf[0] / _gs, _pdf[1] / _gs, _pdf[2] / _gs)


def _sc_body(pinx_h, piny_h, idx4_h, w_h, posx_h, posy_h, nsx_h, nsy_h, mi_h,
             xm_o, xM_o, ym_o, yM_o, hw_o, vw_o,
             i0, i1, i2, i3,
             vx0, vx1, vx2, vx3, vy0, vy1, vy2, vy3, w_v,
             oxm, oxM, oym, oyM, ohw, ovw,
             mi_v, mpx, mpy, msx, msy,
             mxm, mxM, mym, myM, mhw, mvw,
             sem):
    wid = lax.axis_index("s") * 2 + lax.axis_index("c")
    base = wid * _CHUNK
    lane = lax.iota(jnp.int32, 16)

    for k, ib in enumerate((i0, i1, i2, i3)):
        pltpu.sync_copy(idx4_h.at[k, pl.ds(base, _CHUNK)], ib)
    pltpu.sync_copy(w_h.at[pl.ds(base, _CHUNK)], w_v)

    cps = []
    for ib, dst in ((i0, vx0), (i1, vx1), (i2, vx2), (i3, vx3)):
        cps.append(pltpu.async_copy(pinx_h.at[ib], dst, sem))
    for ib, dst in ((i0, vy0), (i1, vy1), (i2, vy2), (i3, vy3)):
        cps.append(pltpu.async_copy(piny_h.at[ib], dst, sem))
    for cp in cps:
        cp.wait()

    def net_step(i, carry):
        s = pl.ds(i * 16, 16)
        a, b, c, d = vx0[s], vx1[s], vx2[s], vx3[s]
        xmin = jnp.minimum(jnp.minimum(a, b), jnp.minimum(c, d))
        xmax = jnp.maximum(jnp.maximum(a, b), jnp.maximum(c, d))
        a, b, c, d = vy0[s], vy1[s], vy2[s], vy3[s]
        ymin = jnp.minimum(jnp.minimum(a, b), jnp.minimum(c, d))
        ymax = jnp.maximum(jnp.maximum(a, b), jnp.maximum(c, d))
        w = w_v[s]
        valid = (base + i * 16 + lane) < _NUM_NETS
        zero = jnp.zeros((16,), jnp.float32)
        hw = jnp.where(valid, w / (ymax - ymin), zero)
        vw = jnp.where(valid, w / (xmax - xmin), zero)
        oxm[s] = jnp.where(valid, xmin, zero)
        oxM[s] = jnp.where(valid, xmax, zero)
        oym[s] = jnp.where(valid, ymin, zero)
        oyM[s] = jnp.where(valid, ymax, zero)
        ohw[s] = hw
        ovw[s] = vw
        return carry

    lax.fori_loop(0, _CHUNK // 16, net_step, 0)

    for src, dst in ((oxm, xm_o), (oxM, xM_o), (oym, ym_o), (oyM, yM_o),
                     (ohw, hw_o), (ovw, vw_o)):
        pltpu.sync_copy(src, dst.at[pl.ds(base, _CHUNK)])

    @pl.when(wid == 0)
    def _macros():
        pltpu.sync_copy(mi_h, mi_v)
        cps2 = []
        for src, dst in ((posx_h, mpx), (posy_h, mpy),
                         (nsx_h, msx), (nsy_h, msy)):
            cps2.append(pltpu.async_copy(src.at[mi_v], dst, sem))
        for cp in cps2:
            cp.wait()

        def macro_step(i, carry):
            s = pl.ds(i * 16, 16)
            px, py, sx, sy = mpx[s], mpy[s], msx[s], msy[s]
            validm = (i * 16 + lane) < _NUM_MACROS
            zero = jnp.zeros((16,), jnp.float32)
            inv_area = 1.0 / (sx * sy)
            mxm[s] = jnp.where(validm, px, zero)
            mxM[s] = jnp.where(validm, px + sx, zero)
            mym[s] = jnp.where(validm, py, zero)
            myM[s] = jnp.where(validm, py + sy, zero)
            mhw[s] = jnp.where(validm, _MACRO_UTIL_H * inv_area, zero)
            mvw[s] = jnp.where(validm, _MACRO_UTIL_V * inv_area, zero)
            return carry

        lax.fori_loop(0, _MACRO_PAD // 16, macro_step, 0)
        for src, dst in ((mxm, xm_o), (mxM, xM_o), (mym, ym_o), (myM, yM_o),
                         (mhw, hw_o), (mvw, vw_o)):
            pltpu.sync_copy(src, dst.at[pl.ds(_NETS_PAD, _MACRO_PAD)])


_sc_kernel = functools.partial(
    pl.kernel,
    out_type=[jax.ShapeDtypeStruct((_KTOT,), jnp.float32)] * 6,
    mesh=plsc.VectorSubcoreMesh(core_axis_name="c", subcore_axis_name="s",
                                num_cores=2, num_subcores=16),
    scratch_types=(
        [pltpu.VMEM((_CHUNK,), jnp.int32)] * 4
        + [pltpu.VMEM((_CHUNK,), jnp.float32)] * 8
        + [pltpu.VMEM((_CHUNK,), jnp.float32)]
        + [pltpu.VMEM((_CHUNK,), jnp.float32)] * 6
        + [pltpu.VMEM((_MACRO_PAD,), jnp.int32)]
        + [pltpu.VMEM((_MACRO_PAD,), jnp.float32)] * 4
        + [pltpu.VMEM((_MACRO_PAD,), jnp.float32)] * 6
        + [pltpu.SemaphoreType.DMA]
    ),
)(_sc_body)


def _blur3(m):
    up = jnp.concatenate([m[1:2, :], m[:-1, :]], axis=0)
    dn = jnp.concatenate([m[1:, :], m[_NBX - 2:_NBX - 1, :]], axis=0)
    t = _G0 * up + _G1 * m + _G2 * dn
    lf = jnp.concatenate([t[:, 1:2], t[:, :-1]], axis=1)
    rt = jnp.concatenate([t[:, 1:], t[:, _NBY - 2:_NBY - 1]], axis=1)
    return _G0 * lf + _G1 * t + _G2 * rt


def _tc_body(xm_ref, xM_ref, hw_ref, vw_ref, ym_ref, yM_ref,
             route_ref, mx_ref, tot_ref, acc_h, acc_v):
    i = pl.program_id(0)

    @pl.when(i == 0)
    def _init():
        acc_h[...] = jnp.zeros((_NBX, _NBY), jnp.float32)
        acc_v[...] = jnp.zeros((_NBX, _NBY), jnp.float32)

    bxl = lax.broadcasted_iota(jnp.int32, (_NBX, 1), 0).astype(jnp.float32) * _BSX
    byl = lax.broadcasted_iota(jnp.int32, (1, _NBY), 1).astype(jnp.float32) * _BSY

    xm = xm_ref[0]
    xM = xM_ref[0]
    hw = hw_ref[0]
    vw = vw_ref[0]
    ym = ym_ref[0]
    yM = yM_ref[0]

    oxt = jnp.maximum(jnp.minimum(xM, bxl + _BSX) - jnp.maximum(xm, bxl), 0.0)
    oy = jnp.maximum(jnp.minimum(yM, byl + _BSY) - jnp.maximum(ym, byl), 0.0)
    acc_h[...] += jnp.dot(oxt * hw, oy, preferred_element_type=jnp.float32)
    acc_v[...] += jnp.dot(oxt * vw, oy, preferred_element_type=jnp.float32)

    @pl.when(i == _G - 1)
    def _finish():
        h = _blur3(acc_h[...] * _INV_CAPA_H)
        v = _blur3(acc_v[...] * _INV_CAPA_V)
        hc = jnp.sum((h > 1.0).astype(jnp.int32))
        vc = jnp.sum((v > 1.0).astype(jnp.int32))
        route_ref[...] = jnp.maximum(jnp.abs(h), jnp.abs(v))
        mx_ref[0, 0] = jnp.maximum(hc, vc)
        tot_ref[0, 0] = hc + vc


def kernel(pos, pin_pos, netpin_start, flat_netpin, net_weights,
           node_size_x, node_size_y, macro_indexes):
    num_nodes = pos.shape[0] // 2
    pin_x = pin_pos[:_NUM_PINS]
    pin_y = pin_pos[_NUM_PINS:]
    pos_x = pos[:num_nodes]
    pos_y = pos[num_nodes:]

    idx4 = jnp.pad(flat_netpin.reshape(_NUM_NETS, _PPN).T,
                   ((0, 0), (0, _NETS_PAD - _NUM_NETS)))
    wpad = jnp.pad(net_weights, (0, _NETS_PAD - _NUM_NETS))
    mpad = jnp.pad(macro_indexes, (0, _MACRO_PAD - _NUM_MACROS))

    xm, xM, ym, yM, hw, vw = _sc_kernel(
        pin_x, pin_y, idx4, wpad, pos_x, pos_y,
        node_size_x, node_size_y, mpad)

    row = lambda a: a.reshape(_G, 1, _KT)
    col = lambda a: a.reshape(_G, _KT, 1)

    row_spec = pl.BlockSpec((1, 1, _KT), lambda i: (i, 0, 0))
    col_spec = pl.BlockSpec((1, _KT, 1), lambda i: (i, 0, 0))
    route, mx, tot = pl.pallas_call(
        _tc_body,
        grid=(_G,),
        in_specs=[row_spec, row_spec, row_spec, row_spec, col_spec, col_spec],
        out_specs=[
            pl.BlockSpec((_NBX, _NBY), lambda i: (0, 0)),
            pl.BlockSpec(memory_space=pltpu.SMEM),
            pl.BlockSpec(memory_space=pltpu.SMEM),
        ],
        out_shape=[
            jax.ShapeDtypeStruct((_NBX, _NBY), jnp.float32),
            jax.ShapeDtypeStruct((1, 1), jnp.int32),
            jax.ShapeDtypeStruct((1, 1), jnp.int32),
        ],
        scratch_shapes=[pltpu.VMEM((_NBX, _NBY), jnp.float32)] * 2,
    )(row(xm), row(xM), row(hw), row(vw), col(ym), col(yM))

    return route, mx.reshape(()), tot.reshape(())


# speedup vs baseline: 62.6240x; 62.6240x over previous
import functools
import math

import jax
import jax.numpy as jnp
from jax import lax
from jax.experimental import pallas as pl
from jax.experimental.pallas import tpu as pltpu
from jax.experimental.pallas import tpu_sc as plsc

_NUM_NETS = 50000
_PPN = 4
_NUM_PINS = _NUM_NETS * _PPN
_NBX = 256
_NBY = 256
_XL, _YL, _XH, _YH = 0.0, 0.0, 1.0, 1.0
_ROUTING_H = 100.0
_ROUTING_V = 100.0
_MACRO_UTIL_H = 10.0
_MACRO_UTIL_V = 10.0
_NUM_MACROS = 200

_NW = 32
_CHUNK = 1568
_NETS_PAD = _NW * _CHUNK
_MACRO_PAD = 512
_KTOT = _NETS_PAD + _MACRO_PAD
_KT = 512
_G = _KTOT // _KT

_BSX = (_XH - _XL) / _NBX
_BSY = (_YH - _YL) / _NBY
_INV_CAPA_H = float(_NBX * _NBY) / _ROUTING_H
_INV_CAPA_V = float(_NBX * _NBY) / _ROUTING_V

_SIGMA = (1.0 / 16.0) * min((_XH - _XL) / _BSX, (_YH - _YL) / _BSY)
_pdf = [math.exp(-0.5 * (t / _SIGMA) ** 2) for t in (-1.0, 0.0, 1.0)]
_gs = sum(_pdf)
_G0, _G1, _G2 = (_pdf[0] / _gs, _pdf[1] / _gs, _pdf[2] / _gs)


def _sc_body(pinx_h, piny_h, ih0, ih1, ih2, ih3, w_h,
             posx_h, posy_h, nsx_h, nsy_h, mi_h,
             xm_o, xM_o, ym_o, yM_o, hw_o, vw_o,
             i0, i1, i2, i3,
             vx0, vx1, vx2, vx3, vy0, vy1, vy2, vy3, w_v,
             oxm, oxM, oym, oyM, ohw, ovw,
             mi_v, mpx, mpy, msx, msy,
             mxm, mxM, mym, myM, mhw, mvw,
             sem):
    wid = lax.axis_index("s") * 2 + lax.axis_index("c")
    base = wid * _CHUNK
    lane = lax.iota(jnp.int32, 16)

    for ih, ib in ((ih0, i0), (ih1, i1), (ih2, i2), (ih3, i3)):
        pltpu.sync_copy(ih.at[pl.ds(base, _CHUNK)], ib)
    pltpu.sync_copy(w_h.at[pl.ds(base, _CHUNK)], w_v)

    cps = []
    for ib, dst in ((i0, vx0), (i1, vx1), (i2, vx2), (i3, vx3)):
        cps.append(pltpu.async_copy(pinx_h.at[ib], dst, sem))
    for ib, dst in ((i0, vy0), (i1, vy1), (i2, vy2), (i3, vy3)):
        cps.append(pltpu.async_copy(piny_h.at[ib], dst, sem))
    for cp in cps:
        cp.wait()

    def net_step(i, carry):
        s = pl.ds(i * 16, 16)
        a, b, c, d = vx0[s], vx1[s], vx2[s], vx3[s]
        xmin = jnp.minimum(jnp.minimum(a, b), jnp.minimum(c, d))
        xmax = jnp.maximum(jnp.maximum(a, b), jnp.maximum(c, d))
        a, b, c, d = vy0[s], vy1[s], vy2[s], vy3[s]
        ymin = jnp.minimum(jnp.minimum(a, b), jnp.minimum(c, d))
        ymax = jnp.maximum(jnp.maximum(a, b), jnp.maximum(c, d))
        w = w_v[s]
        valid = (base + i * 16 + lane) < _NUM_NETS
        zero = jnp.zeros((16,), jnp.float32)
        hw = jnp.where(valid, w / (ymax - ymin), zero)
        vw = jnp.where(valid, w / (xmax - xmin), zero)
        oxm[s] = jnp.where(valid, xmin, zero)
        oxM[s] = jnp.where(valid, xmax, zero)
        oym[s] = jnp.where(valid, ymin, zero)
        oyM[s] = jnp.where(valid, ymax, zero)
        ohw[s] = hw
        ovw[s] = vw
        return carry

    lax.fori_loop(0, _CHUNK // 16, net_step, 0)

    for src, dst in ((oxm, xm_o), (oxM, xM_o), (oym, ym_o), (oyM, yM_o),
                     (ohw, hw_o), (ovw, vw_o)):
        pltpu.sync_copy(src, dst.at[pl.ds(base, _CHUNK)])

    @pl.when(wid == 0)
    def _macros():
        pltpu.sync_copy(mi_h, mi_v)
        cps2 = []
        for src, dst in ((posx_h, mpx), (posy_h, mpy),
                         (nsx_h, msx), (nsy_h, msy)):
            cps2.append(pltpu.async_copy(src.at[mi_v], dst, sem))
        for cp in cps2:
            cp.wait()

        def macro_step(i, carry):
            s = pl.ds(i * 16, 16)
            px, py, sx, sy = mpx[s], mpy[s], msx[s], msy[s]
            validm = (i * 16 + lane) < _NUM_MACROS
            zero = jnp.zeros((16,), jnp.float32)
            inv_area = 1.0 / (sx * sy)
            mxm[s] = jnp.where(validm, px, zero)
            mxM[s] = jnp.where(validm, px + sx, zero)
            mym[s] = jnp.where(validm, py, zero)
            myM[s] = jnp.where(validm, py + sy, zero)
            mhw[s] = jnp.where(validm, _MACRO_UTIL_H * inv_area, zero)
            mvw[s] = jnp.where(validm, _MACRO_UTIL_V * inv_area, zero)
            return carry

        lax.fori_loop(0, _MACRO_PAD // 16, macro_step, 0)
        for src, dst in ((mxm, xm_o), (mxM, xM_o), (mym, ym_o), (myM, yM_o),
                         (mhw, hw_o), (mvw, vw_o)):
            pltpu.sync_copy(src, dst.at[pl.ds(_NETS_PAD, _MACRO_PAD)])


@functools.lru_cache(maxsize=1)
def _make_sc_kernel():
  return functools.partial(
    pl.kernel,
    out_type=[jax.ShapeDtypeStruct((_KTOT,), jnp.float32)] * 6,
    mesh=plsc.VectorSubcoreMesh(core_axis_name="c", subcore_axis_name="s",
                                num_cores=2, num_subcores=16),
    scratch_types=(
        [pltpu.VMEM((_CHUNK,), jnp.int32)] * 4
        + [pltpu.VMEM((_CHUNK,), jnp.float32)] * 8
        + [pltpu.VMEM((_CHUNK,), jnp.float32)]
        + [pltpu.VMEM((_CHUNK,), jnp.float32)] * 6
        + [pltpu.VMEM((_MACRO_PAD,), jnp.int32)]
        + [pltpu.VMEM((_MACRO_PAD,), jnp.float32)] * 4
        + [pltpu.VMEM((_MACRO_PAD,), jnp.float32)] * 6
        + [pltpu.SemaphoreType.DMA]
    ),
  )(_sc_body)


def _blur3(m):
    up = jnp.concatenate([m[1:2, :], m[:-1, :]], axis=0)
    dn = jnp.concatenate([m[1:, :], m[_NBX - 2:_NBX - 1, :]], axis=0)
    t = _G0 * up + _G1 * m + _G2 * dn
    lf = jnp.concatenate([t[:, 1:2], t[:, :-1]], axis=1)
    rt = jnp.concatenate([t[:, 1:], t[:, _NBY - 2:_NBY - 1]], axis=1)
    return _G0 * lf + _G1 * t + _G2 * rt


def _tc_body(xm_ref, xM_ref, hw_ref, vw_ref, ym_ref, yM_ref,
             route_ref, mx_ref, tot_ref, acc_h, acc_v):
    i = pl.program_id(0)

    @pl.when(i == 0)
    def _init():
        acc_h[...] = jnp.zeros((_NBX, _NBY), jnp.float32)
        acc_v[...] = jnp.zeros((_NBX, _NBY), jnp.float32)

    bxl = lax.broadcasted_iota(jnp.int32, (_NBX, 1), 0).astype(jnp.float32) * _BSX
    byl = lax.broadcasted_iota(jnp.int32, (1, _NBY), 1).astype(jnp.float32) * _BSY

    xm = xm_ref[0]
    xM = xM_ref[0]
    hw = hw_ref[0]
    vw = vw_ref[0]
    ym = ym_ref[0]
    yM = yM_ref[0]

    oxt = jnp.maximum(jnp.minimum(xM, bxl + _BSX) - jnp.maximum(xm, bxl), 0.0)
    oy = jnp.maximum(jnp.minimum(yM, byl + _BSY) - jnp.maximum(ym, byl), 0.0)
    acc_h[...] += jnp.dot(oxt * hw, oy, preferred_element_type=jnp.float32)
    acc_v[...] += jnp.dot(oxt * vw, oy, preferred_element_type=jnp.float32)

    @pl.when(i == _G - 1)
    def _finish():
        h = _blur3(acc_h[...] * _INV_CAPA_H)
        v = _blur3(acc_v[...] * _INV_CAPA_V)
        hc = jnp.sum((h > 1.0).astype(jnp.int32))
        vc = jnp.sum((v > 1.0).astype(jnp.int32))
        route_ref[...] = jnp.maximum(jnp.abs(h), jnp.abs(v))
        mx_ref[0, 0] = jnp.maximum(hc, vc)
        tot_ref[0, 0] = hc + vc


def kernel(pos, pin_pos, netpin_start, flat_netpin, net_weights,
           node_size_x, node_size_y, macro_indexes):
    num_nodes = pos.shape[0] // 2
    pin_x = pin_pos[:_NUM_PINS]
    pin_y = pin_pos[_NUM_PINS:]
    pos_x = pos[:num_nodes]
    pos_y = pos[num_nodes:]

    fn = flat_netpin.reshape(_NUM_NETS, _PPN)
    idx4 = [jnp.pad(fn[:, k], (0, _NETS_PAD - _NUM_NETS)) for k in range(_PPN)]
    wpad = jnp.pad(net_weights, (0, _NETS_PAD - _NUM_NETS))
    mpad = jnp.pad(macro_indexes, (0, _MACRO_PAD - _NUM_MACROS))

    xm, xM, ym, yM, hw, vw = _make_sc_kernel()(
        pin_x, pin_y, idx4[0], idx4[1], idx4[2], idx4[3], wpad, pos_x, pos_y,
        node_size_x, node_size_y, mpad)

    row = lambda a: a.reshape(_G, 1, _KT)
    col = lambda a: a.reshape(_G, _KT, 1)

    row_spec = pl.BlockSpec((1, 1, _KT), lambda i: (i, 0, 0))
    col_spec = pl.BlockSpec((1, _KT, 1), lambda i: (i, 0, 0))
    route, mx, tot = pl.pallas_call(
        _tc_body,
        grid=(_G,),
        in_specs=[row_spec, row_spec, row_spec, row_spec, col_spec, col_spec],
        out_specs=[
            pl.BlockSpec((_NBX, _NBY), lambda i: (0, 0)),
            pl.BlockSpec(memory_space=pltpu.SMEM),
            pl.BlockSpec(memory_space=pltpu.SMEM),
        ],
        out_shape=[
            jax.ShapeDtypeStruct((_NBX, _NBY), jnp.float32),
            jax.ShapeDtypeStruct((1, 1), jnp.int32),
            jax.ShapeDtypeStruct((1, 1), jnp.int32),
        ],
        scratch_shapes=[pltpu.VMEM((_NBX, _NBY), jnp.float32)] * 2,
    )(row(xm), row(xM), row(hw), row(vw), col(ym), col(yM))

    return route, mx.reshape(()), tot.reshape(())


# speedup vs baseline: 97.8880x; 1.5631x over previous
import functools
import math

import jax
import jax.numpy as jnp
from jax import lax
from jax.experimental import pallas as pl
from jax.experimental.pallas import tpu as pltpu
from jax.experimental.pallas import tpu_sc as plsc

_NUM_NETS = 50000
_PPN = 4
_NUM_PINS = _NUM_NETS * _PPN
_NBX = 256
_NBY = 256
_XL, _YL, _XH, _YH = 0.0, 0.0, 1.0, 1.0
_ROUTING_H = 100.0
_ROUTING_V = 100.0
_MACRO_UTIL_H = 10.0
_MACRO_UTIL_V = 10.0
_NUM_MACROS = 200

_NW = 32
_CHUNK = 1568
_NETS_PAD = _NW * _CHUNK
_MACRO_PAD = 1024
_KTOT = _NETS_PAD + _MACRO_PAD
_KT = 1024
_G = _KTOT // _KT

_BSX = (_XH - _XL) / _NBX
_BSY = (_YH - _YL) / _NBY
_INV_CAPA_H = float(_NBX * _NBY) / _ROUTING_H
_INV_CAPA_V = float(_NBX * _NBY) / _ROUTING_V

_SIGMA = (1.0 / 16.0) * min((_XH - _XL) / _BSX, (_YH - _YL) / _BSY)
_pdf = [math.exp(-0.5 * (t / _SIGMA) ** 2) for t in (-1.0, 0.0, 1.0)]
_gs = sum(_pdf)
_G0, _G1, _G2 = (_pdf[0] / _gs, _pdf[1] / _gs, _pdf[2] / _gs)


def _sc_body(pinx_h, piny_h, ih0, ih1, ih2, ih3, w_h,
             posx_h, posy_h, nsx_h, nsy_h, mi_h,
             xm_o, xM_o, ym_o, yM_o, hw_o, vw_o,
             i0, i1, i2, i3,
             vx0, vx1, vx2, vx3, vy0, vy1, vy2, vy3, w_v,
             oxm, oxM, oym, oyM, ohw, ovw,
             mi_v, mpx, mpy, msx, msy,
             mxm, mxM, mym, myM, mhw, mvw,
             sem):
    wid = lax.axis_index("s") * 2 + lax.axis_index("c")
    base = wid * _CHUNK
    lane = lax.iota(jnp.int32, 16)

    for ih, ib in ((ih0, i0), (ih1, i1), (ih2, i2), (ih3, i3)):
        pltpu.sync_copy(ih.at[pl.ds(base, _CHUNK)], ib)
    pltpu.sync_copy(w_h.at[pl.ds(base, _CHUNK)], w_v)

    cps = []
    for ib, dst in ((i0, vx0), (i1, vx1), (i2, vx2), (i3, vx3)):
        cps.append(pltpu.async_copy(pinx_h.at[ib], dst, sem))
    for ib, dst in ((i0, vy0), (i1, vy1), (i2, vy2), (i3, vy3)):
        cps.append(pltpu.async_copy(piny_h.at[ib], dst, sem))
    for cp in cps:
        cp.wait()

    def net_step(i, carry):
        s = pl.ds(i * 16, 16)
        a, b, c, d = vx0[s], vx1[s], vx2[s], vx3[s]
        xmin = jnp.minimum(jnp.minimum(a, b), jnp.minimum(c, d))
        xmax = jnp.maximum(jnp.maximum(a, b), jnp.maximum(c, d))
        a, b, c, d = vy0[s], vy1[s], vy2[s], vy3[s]
        ymin = jnp.minimum(jnp.minimum(a, b), jnp.minimum(c, d))
        ymax = jnp.maximum(jnp.maximum(a, b), jnp.maximum(c, d))
        w = w_v[s]
        valid = (base + i * 16 + lane) < _NUM_NETS
        zero = jnp.zeros((16,), jnp.float32)
        hw = jnp.where(valid, w / (ymax - ymin), zero)
        vw = jnp.where(valid, w / (xmax - xmin), zero)
        oxm[s] = jnp.where(valid, xmin, zero)
        oxM[s] = jnp.where(valid, xmax, zero)
        oym[s] = jnp.where(valid, ymin, zero)
        oyM[s] = jnp.where(valid, ymax, zero)
        ohw[s] = hw
        ovw[s] = vw
        return carry

    lax.fori_loop(0, _CHUNK // 16, net_step, 0)

    for src, dst in ((oxm, xm_o), (oxM, xM_o), (oym, ym_o), (oyM, yM_o),
                     (ohw, hw_o), (ovw, vw_o)):
        pltpu.sync_copy(src, dst.at[pl.ds(base, _CHUNK)])

    @pl.when(wid == 0)
    def _macros():
        pltpu.sync_copy(mi_h, mi_v)
        cps2 = []
        for src, dst in ((posx_h, mpx), (posy_h, mpy),
                         (nsx_h, msx), (nsy_h, msy)):
            cps2.append(pltpu.async_copy(src.at[mi_v], dst, sem))
        for cp in cps2:
            cp.wait()

        def macro_step(i, carry):
            s = pl.ds(i * 16, 16)
            px, py, sx, sy = mpx[s], mpy[s], msx[s], msy[s]
            validm = (i * 16 + lane) < _NUM_MACROS
            zero = jnp.zeros((16,), jnp.float32)
            inv_area = 1.0 / (sx * sy)
            mxm[s] = jnp.where(validm, px, zero)
            mxM[s] = jnp.where(validm, px + sx, zero)
            mym[s] = jnp.where(validm, py, zero)
            myM[s] = jnp.where(validm, py + sy, zero)
            mhw[s] = jnp.where(validm, _MACRO_UTIL_H * inv_area, zero)
            mvw[s] = jnp.where(validm, _MACRO_UTIL_V * inv_area, zero)
            return carry

        lax.fori_loop(0, _MACRO_PAD // 16, macro_step, 0)
        for src, dst in ((mxm, xm_o), (mxM, xM_o), (mym, ym_o), (myM, yM_o),
                         (mhw, hw_o), (mvw, vw_o)):
            pltpu.sync_copy(src, dst.at[pl.ds(_NETS_PAD, _MACRO_PAD)])


@functools.lru_cache(maxsize=1)
def _make_sc_kernel():
  return functools.partial(
    pl.kernel,
    out_type=[jax.ShapeDtypeStruct((_KTOT,), jnp.float32)] * 6,
    mesh=plsc.VectorSubcoreMesh(core_axis_name="c", subcore_axis_name="s",
                                num_cores=2, num_subcores=16),
    scratch_types=(
        [pltpu.VMEM((_CHUNK,), jnp.int32)] * 4
        + [pltpu.VMEM((_CHUNK,), jnp.float32)] * 8
        + [pltpu.VMEM((_CHUNK,), jnp.float32)]
        + [pltpu.VMEM((_CHUNK,), jnp.float32)] * 6
        + [pltpu.VMEM((_MACRO_PAD,), jnp.int32)]
        + [pltpu.VMEM((_MACRO_PAD,), jnp.float32)] * 4
        + [pltpu.VMEM((_MACRO_PAD,), jnp.float32)] * 6
        + [pltpu.SemaphoreType.DMA]
    ),
  )(_sc_body)


def _blur3(m):
    up = jnp.concatenate([m[1:2, :], m[:-1, :]], axis=0)
    dn = jnp.concatenate([m[1:, :], m[_NBX - 2:_NBX - 1, :]], axis=0)
    t = _G0 * up + _G1 * m + _G2 * dn
    lf = jnp.concatenate([t[:, 1:2], t[:, :-1]], axis=1)
    rt = jnp.concatenate([t[:, 1:], t[:, _NBY - 2:_NBY - 1]], axis=1)
    return _G0 * lf + _G1 * t + _G2 * rt


def _tc_body(xm_ref, xM_ref, hw_ref, vw_ref, ym_ref, yM_ref,
             route_ref, mx_ref, tot_ref, acc_h, acc_v):
    i = pl.program_id(0)

    @pl.when(i == 0)
    def _init():
        acc_h[...] = jnp.zeros((_NBX, _NBY), jnp.float32)
        acc_v[...] = jnp.zeros((_NBX, _NBY), jnp.float32)

    bxl = lax.broadcasted_iota(jnp.int32, (_NBX, 1), 0).astype(jnp.float32) * _BSX

    xm = xm_ref[0]
    xM = xM_ref[0]
    hw = hw_ref[0]
    vw = vw_ref[0]
    ym = ym_ref[0]
    yM = yM_ref[0]

    oxt = jnp.maximum(jnp.minimum(xM, bxl + _BSX) - jnp.maximum(xm, bxl), 0.0)
    oyt = jnp.maximum(jnp.minimum(yM, bxl + _BSY) - jnp.maximum(ym, bxl), 0.0)
    oyt_bf = oyt.astype(jnp.bfloat16)
    dn = (((1,), (1,)), ((), ()))
    acc_h[...] += lax.dot_general((oxt * hw).astype(jnp.bfloat16), oyt_bf,
                                  dn, preferred_element_type=jnp.float32)
    acc_v[...] += lax.dot_general((oxt * vw).astype(jnp.bfloat16), oyt_bf,
                                  dn, preferred_element_type=jnp.float32)

    @pl.when(i == _G - 1)
    def _finish():
        h = _blur3(acc_h[...] * _INV_CAPA_H)
        v = _blur3(acc_v[...] * _INV_CAPA_V)
        hc = jnp.sum((h > 1.0).astype(jnp.int32))
        vc = jnp.sum((v > 1.0).astype(jnp.int32))
        route_ref[...] = jnp.maximum(jnp.abs(h), jnp.abs(v))
        mx_ref[0, 0] = jnp.maximum(hc, vc)
        tot_ref[0, 0] = hc + vc


def kernel(pos, pin_pos, netpin_start, flat_netpin, net_weights,
           node_size_x, node_size_y, macro_indexes):
    num_nodes = pos.shape[0] // 2
    pin_x = pin_pos[:_NUM_PINS]
    pin_y = pin_pos[_NUM_PINS:]
    pos_x = pos[:num_nodes]
    pos_y = pos[num_nodes:]

    fn = flat_netpin.reshape(_NUM_NETS, _PPN)
    idx4 = [jnp.pad(fn[:, k], (0, _NETS_PAD - _NUM_NETS)) for k in range(_PPN)]
    wpad = jnp.pad(net_weights, (0, _NETS_PAD - _NUM_NETS))
    mpad = jnp.pad(macro_indexes, (0, _MACRO_PAD - _NUM_MACROS))

    xm, xM, ym, yM, hw, vw = _make_sc_kernel()(
        pin_x, pin_y, idx4[0], idx4[1], idx4[2], idx4[3], wpad, pos_x, pos_y,
        node_size_x, node_size_y, mpad)

    row = lambda a: a.reshape(_G, 1, _KT)

    row_spec = pl.BlockSpec((1, 1, _KT), lambda i: (i, 0, 0))
    route, mx, tot = pl.pallas_call(
        _tc_body,
        grid=(_G,),
        in_specs=[row_spec] * 6,
        out_specs=[
            pl.BlockSpec((_NBX, _NBY), lambda i: (0, 0)),
            pl.BlockSpec(memory_space=pltpu.SMEM),
            pl.BlockSpec(memory_space=pltpu.SMEM),
        ],
        out_shape=[
            jax.ShapeDtypeStruct((_NBX, _NBY), jnp.float32),
            jax.ShapeDtypeStruct((1, 1), jnp.int32),
            jax.ShapeDtypeStruct((1, 1), jnp.int32),
        ],
        scratch_shapes=[pltpu.VMEM((_NBX, _NBY), jnp.float32)] * 2,
    )(row(xm), row(xM), row(hw), row(vw), row(ym), row(yM))

    return route, mx.reshape(()), tot.reshape(())
